# Initial kernel scaffold; baseline (speedup 1.0000x reference)
import jax
import jax.numpy as jnp
from jax.experimental import pallas as pl


def kernel(x, edge_index, W_msg, Wz, Uz, bz, Wr, Ur, br, Wh, Uh, bh):
    raise NotImplementedError("write your pallas kernel here")


# speedup vs baseline: 3.6671x; 3.6671x over previous
import functools

import jax
import jax.numpy as jnp
from jax import lax
from jax.experimental import pallas as pl
from jax.experimental.pallas import tpu as pltpu
from jax.experimental.pallas import tpu_sc as plsc

N = 10000
D = 128
NC = 2
NS = 16
L = 16
HALF = N // NC
ACC_ROWS = 5120
PAD_DST = N
CHUNK = 128
C_PER_TILE = 158
EP = NS * C_PER_TILE * CHUNK
ZROWS_PER_TILE = ACC_ROWS // NS
ZR = 128

R = 1000
GRID = N // R


def _seg_sum_body(m, eidx, zrows, out, src_v, dst_v, gb0, gb1, acc,
                  sem0, sem1):
    c = lax.axis_index("c")
    s = lax.axis_index("s")
    pltpu.sync_copy(eidx.at[0].at[s], src_v)
    pltpu.sync_copy(eidx.at[1].at[s], dst_v)
    zbase = s * ZROWS_PER_TILE
    for k in range(ZROWS_PER_TILE // ZR):
        pltpu.sync_copy(zrows, acc.at[pl.ds(zbase + k * ZR, ZR)])
    zrem = ZROWS_PER_TILE % ZR
    if zrem:
        pltpu.sync_copy(zrows.at[pl.ds(0, zrem)],
                        acc.at[pl.ds(zbase + (ZROWS_PER_TILE // ZR) * ZR,
                                     zrem)])
    lo = (c * HALF).astype(jnp.int32)
    dump = jnp.full((L,), HALF, dtype=jnp.int32)

    def remap(j, carry):
        for k in range(CHUNK // L):
            sl = pl.ds(k * L, L)
            d = dst_v[j, sl]
            loc = d - lo
            ok = (d >= lo) & (loc < HALF)
            dst_v[j, sl] = jnp.where(ok, loc, dump)
        return carry

    lax.fori_loop(0, C_PER_TILE, remap, 0)
    plsc.subcore_barrier()

    pltpu.async_copy(m.at[src_v.at[0]], gb0, sem0)

    def body(j, carry):
        j0 = 2 * j
        j1 = j0 + 1
        pltpu.async_copy(m.at[src_v.at[j1]], gb1, sem1)
        pltpu.make_async_copy(m.at[src_v.at[j0]], gb0, sem0).wait()
        pltpu.sync_copy(gb0, acc.at[dst_v.at[j0]], add=True)
        jn = jnp.minimum(j0 + 2, C_PER_TILE - 1)
        pltpu.async_copy(m.at[src_v.at[jn]], gb0, sem0)
        pltpu.make_async_copy(m.at[src_v.at[j1]], gb1, sem1).wait()
        pltpu.sync_copy(gb1, acc.at[dst_v.at[j1]], add=True)
        return carry

    lax.fori_loop(0, C_PER_TILE // 2, body, 0)
    pltpu.make_async_copy(m.at[src_v.at[0]], gb0, sem0).wait()
    plsc.subcore_barrier()
    obase = s * ZROWS_PER_TILE
    orows = pl.ds(obase, ZROWS_PER_TILE)
    pltpu.sync_copy(acc.at[orows], out.at[c].at[orows])


_seg_sum = functools.partial(
    pl.kernel,
    out_type=jax.ShapeDtypeStruct((NC, ACC_ROWS, D), jnp.float32),
    mesh=plsc.VectorSubcoreMesh(core_axis_name="c", subcore_axis_name="s",
                                num_cores=NC, num_subcores=NS),
    scratch_types=[
        pltpu.VMEM((C_PER_TILE, CHUNK), jnp.int32),
        pltpu.VMEM((C_PER_TILE, CHUNK), jnp.int32),
        pltpu.VMEM((CHUNK, D), jnp.float32),
        pltpu.VMEM((CHUNK, D), jnp.float32),
        pltpu.VMEM_SHARED((ACC_ROWS, D), jnp.float32),
        pltpu.SemaphoreType.DMA,
        pltpu.SemaphoreType.DMA,
    ],
)(_seg_sum_body)


def _w_spec():
    return pl.BlockSpec((D, D), lambda i: (0, 0))


def _b_spec():
    return pl.BlockSpec((1, D), lambda i: (0, 0))


def _gru_first():
    def body(x, Wz, Wh, Wm, bz, bh, h_out, m_out):
        a = x[0]
        z = jax.nn.sigmoid(jnp.dot(a, Wz[...]) + bz[...])
        hc = jnp.tanh(jnp.dot(a, Wh[...]) + bh[...])
        h = z * hc
        h_out[...] = h
        m_out[...] = jnp.dot(h, Wm[...])

    return pl.pallas_call(
        body,
        grid=(GRID,),
        in_specs=[
            pl.BlockSpec((1, R, D), lambda i: (0, i, 0)),
            _w_spec(), _w_spec(), _w_spec(), _b_spec(), _b_spec(),
        ],
        out_specs=[
            pl.BlockSpec((R, D), lambda i: (i, 0)),
            pl.BlockSpec((R, D), lambda i: (i, 0)),
        ],
        out_shape=[
            jax.ShapeDtypeStruct((N, D), jnp.float32),
            jax.ShapeDtypeStruct((N, D), jnp.float32),
        ],
    )


_BLOCKS_PER_HALF = HALF // R


def _gru_step(t, emit_m):
    def body(x, agg, h_in, Wz, Uz, Wr, Ur, Wh, Uh, Wm, bz, br, bh,
             h_out, *maybe_m):
        a = x[0] + agg[0]
        h = h_in[...]
        z = jax.nn.sigmoid(jnp.dot(a, Wz[...]) + jnp.dot(h, Uz[...]) + bz[...])
        r = jax.nn.sigmoid(jnp.dot(a, Wr[...]) + jnp.dot(h, Ur[...]) + br[...])
        hc = jnp.tanh(jnp.dot(a, Wh[...]) + jnp.dot(r * h, Uh[...]) + bh[...])
        hn = (1.0 - z) * h + z * hc
        h_out[...] = hn
        if maybe_m:
            maybe_m[0][...] = jnp.dot(hn, Wm[...])

    out_specs = [pl.BlockSpec((R, D), lambda i: (i, 0))]
    out_shape = [jax.ShapeDtypeStruct((N, D), jnp.float32)]
    if emit_m:
        out_specs.append(pl.BlockSpec((R, D), lambda i: (i, 0)))
        out_shape.append(jax.ShapeDtypeStruct((N, D), jnp.float32))

    return pl.pallas_call(
        body,
        grid=(GRID,),
        in_specs=[
            pl.BlockSpec((1, R, D), lambda i, t=t: (t, i, 0)),
            pl.BlockSpec((1, R, D),
                         lambda i: (i // _BLOCKS_PER_HALF,
                                    i % _BLOCKS_PER_HALF, 0)),
            pl.BlockSpec((R, D), lambda i: (i, 0)),
            _w_spec(), _w_spec(), _w_spec(), _w_spec(), _w_spec(), _w_spec(),
            _w_spec(), _b_spec(), _b_spec(), _b_spec(),
        ],
        out_specs=out_specs,
        out_shape=out_shape,
    )


def kernel(x, edge_index, W_msg, Wz, Uz, bz, Wr, Ur, br, Wh, Uh, bh):
    T = x.shape[0]
    src = edge_index[0]
    dst = edge_index[1]
    pad = EP - src.shape[0]
    src_p = jnp.concatenate([src, jnp.zeros((pad,), jnp.int32)])
    dst_p = jnp.concatenate([dst, jnp.full((pad,), PAD_DST, jnp.int32)])
    eidx = jnp.stack([src_p, dst_p]).reshape(2, NS, C_PER_TILE, CHUNK)
    zrows = jnp.zeros((ZR, D), jnp.float32)
    bz2 = bz.reshape(1, D)
    br2 = br.reshape(1, D)
    bh2 = bh.reshape(1, D)

    h, m = _gru_first()(x, Wz, Wh, W_msg, bz2, bh2)
    for t in range(1, T):
        agg = _seg_sum(m, eidx, zrows)
        outs = _gru_step(t, emit_m=(t < T - 1))(
            x, agg, h, Wz, Uz, Wr, Ur, Wh, Uh, W_msg, bz2, br2, bh2)
        if t < T - 1:
            h, m = outs
        else:
            h = outs[0]
    return h


# speedup vs baseline: 3.8916x; 1.0612x over previous
import functools

import jax
import jax.numpy as jnp
from jax import lax
from jax.experimental import pallas as pl
from jax.experimental.pallas import tpu as pltpu
from jax.experimental.pallas import tpu_sc as plsc

N = 10000
D = 128
NC = 2
NS = 16
NW = NC * NS
CHUNK = 128
C2 = 80
EP = NW * C2 * CHUNK
RB = 16
NBATCH = C2 // RB
NBUF = 2
ACC_ROWS = 10240
ZROWS_PER_TILE = ACC_ROWS // NS
DUMP_LO = N + 8
ZR = 128

R = 1000
GRID = N // R


def _seg_sum_body(m, eidx, zrows, out, sr0, dr0, sr1, dr1, gb0, gb1, acc,
                  g0, g1, i0, i1):
    c = lax.axis_index("c")
    s = lax.axis_index("s")
    w = c * NS + s
    srs = (sr0, sr1)
    drs = (dr0, dr1)
    isem = (i0, i1)
    gbs = (gb0, gb1)
    gsem = (g0, g1)

    def fire_idx(bb, r):
        sl = pl.ds(bb * RB, RB)
        pltpu.async_copy(eidx.at[0].at[w].at[sl], srs[r], isem[r])
        pltpu.async_copy(eidx.at[1].at[w].at[sl], drs[r], isem[r])

    def wait_idx(r):
        sl = pl.ds(0, RB)
        pltpu.make_async_copy(eidx.at[0].at[w].at[sl], srs[r], isem[r]).wait()
        pltpu.make_async_copy(eidx.at[1].at[w].at[sl], drs[r], isem[r]).wait()

    fire_idx(0, 0)
    fire_idx(1, 1)
    zbase = s * ZROWS_PER_TILE
    for k in range(ZROWS_PER_TILE // ZR):
        pltpu.sync_copy(zrows, acc.at[pl.ds(zbase + k * ZR, ZR)])
    plsc.subcore_barrier()

    for bb in range(NBATCH):
        r = bb % 2
        wait_idx(r)
        sr = srs[r]
        dr = drs[r]
        for b in range(NBUF):
            pltpu.async_copy(m.at[sr.at[b]], gbs[b], gsem[b])

        def body(jj, carry, sr=sr, dr=dr):
            c0 = NBUF * jj
            for b in range(NBUF):
                ch = c0 + b
                pltpu.make_async_copy(m.at[sr.at[0]], gbs[b], gsem[b]).wait()
                pltpu.sync_copy(gbs[b], acc.at[dr.at[ch]], add=True)
                pltpu.async_copy(m.at[sr.at[ch + NBUF]], gbs[b], gsem[b])
            return carry

        lax.fori_loop(0, RB // NBUF - 1, body, 0)
        for b in range(NBUF):
            ch = RB - NBUF + b
            pltpu.make_async_copy(m.at[sr.at[0]], gbs[b], gsem[b]).wait()
            pltpu.sync_copy(gbs[b], acc.at[dr.at[ch]], add=True)
        if bb + 2 < NBATCH:
            fire_idx(bb + 2, r)

    plsc.subcore_barrier()
    orows = pl.ds(s * ZROWS_PER_TILE, ZROWS_PER_TILE)
    pltpu.sync_copy(acc.at[orows], out.at[c].at[orows])


_seg_sum = functools.partial(
    pl.kernel,
    out_type=jax.ShapeDtypeStruct((NC, ACC_ROWS, D), jnp.float32),
    mesh=plsc.VectorSubcoreMesh(core_axis_name="c", subcore_axis_name="s",
                                num_cores=NC, num_subcores=NS),
    scratch_types=[
        pltpu.VMEM((RB, CHUNK), jnp.int32),
        pltpu.VMEM((RB, CHUNK), jnp.int32),
        pltpu.VMEM((RB, CHUNK), jnp.int32),
        pltpu.VMEM((RB, CHUNK), jnp.int32),
        pltpu.VMEM((CHUNK, D), jnp.float32),
        pltpu.VMEM((CHUNK, D), jnp.float32),
        pltpu.VMEM_SHARED((ACC_ROWS, D), jnp.float32),
        pltpu.SemaphoreType.DMA,
        pltpu.SemaphoreType.DMA,
        pltpu.SemaphoreType.DMA,
        pltpu.SemaphoreType.DMA,
    ],
)(_seg_sum_body)


def _w_spec():
    return pl.BlockSpec((D, D), lambda i: (0, 0))


def _b_spec():
    return pl.BlockSpec((1, D), lambda i: (0, 0))


def _gru_first():
    def body(x, Wz, Wh, Wm, bz, bh, h_out, m_out):
        a = x[0]
        z = jax.nn.sigmoid(jnp.dot(a, Wz[...]) + bz[...])
        hc = jnp.tanh(jnp.dot(a, Wh[...]) + bh[...])
        h = z * hc
        h_out[...] = h
        m_out[...] = jnp.dot(h, Wm[...])

    return pl.pallas_call(
        body,
        grid=(GRID,),
        in_specs=[
            pl.BlockSpec((1, R, D), lambda i: (0, i, 0)),
            _w_spec(), _w_spec(), _w_spec(), _b_spec(), _b_spec(),
        ],
        out_specs=[
            pl.BlockSpec((R, D), lambda i: (i, 0)),
            pl.BlockSpec((R, D), lambda i: (i, 0)),
        ],
        out_shape=[
            jax.ShapeDtypeStruct((N, D), jnp.float32),
            jax.ShapeDtypeStruct((N, D), jnp.float32),
        ],
    )


def _gru_step(t, emit_m):
    def body(x, agg, h_in, Wz, Uz, Wr, Ur, Wh, Uh, Wm, bz, br, bh,
             h_out, *maybe_m):
        a = x[0] + agg[0] + agg[1]
        h = h_in[...]
        z = jax.nn.sigmoid(jnp.dot(a, Wz[...]) + jnp.dot(h, Uz[...]) + bz[...])
        r = jax.nn.sigmoid(jnp.dot(a, Wr[...]) + jnp.dot(h, Ur[...]) + br[...])
        hc = jnp.tanh(jnp.dot(a, Wh[...]) + jnp.dot(r * h, Uh[...]) + bh[...])
        hn = (1.0 - z) * h + z * hc
        h_out[...] = hn
        if maybe_m:
            maybe_m[0][...] = jnp.dot(hn, Wm[...])

    out_specs = [pl.BlockSpec((R, D), lambda i: (i, 0))]
    out_shape = [jax.ShapeDtypeStruct((N, D), jnp.float32)]
    if emit_m:
        out_specs.append(pl.BlockSpec((R, D), lambda i: (i, 0)))
        out_shape.append(jax.ShapeDtypeStruct((N, D), jnp.float32))

    return pl.pallas_call(
        body,
        grid=(GRID,),
        in_specs=[
            pl.BlockSpec((1, R, D), lambda i, t=t: (t, i, 0)),
            pl.BlockSpec((NC, R, D), lambda i: (0, i, 0)),
            pl.BlockSpec((R, D), lambda i: (i, 0)),
            _w_spec(), _w_spec(), _w_spec(), _w_spec(), _w_spec(), _w_spec(),
            _w_spec(), _b_spec(), _b_spec(), _b_spec(),
        ],
        out_specs=out_specs,
        out_shape=out_shape,
    )


def kernel(x, edge_index, W_msg, Wz, Uz, bz, Wr, Ur, br, Wh, Uh, bh):
    T = x.shape[0]
    src = edge_index[0]
    dst = edge_index[1]
    pad = EP - src.shape[0]
    src_p = jnp.concatenate([src, jnp.zeros((pad,), jnp.int32)])
    dump = DUMP_LO + (jnp.arange(pad, dtype=jnp.int32) % (ACC_ROWS - DUMP_LO))
    dst_p = jnp.concatenate([dst, dump])
    eidx = jnp.stack([src_p, dst_p]).reshape(2, NW, C2, CHUNK)
    zrows = jnp.zeros((ZR, D), jnp.float32)
    bz2 = bz.reshape(1, D)
    br2 = br.reshape(1, D)
    bh2 = bh.reshape(1, D)

    h, m = _gru_first()(x, Wz, Wh, W_msg, bz2, bh2)
    for t in range(1, T):
        agg = _seg_sum(m, eidx, zrows)
        outs = _gru_step(t, emit_m=(t < T - 1))(
            x, agg, h, Wz, Uz, Wr, Ur, Wh, Uh, W_msg, bz2, br2, bh2)
        if t < T - 1:
            h, m = outs
        else:
            h = outs[0]
    return h
